# baseline (speedup 1.0000x reference)
import math

import jax
import jax.numpy as jnp
from jax import lax
from jax.experimental import pallas as pl
from jax.experimental.pallas import tpu as pltpu
from jax.experimental.pallas import tpu_sc as plsc

_ALPHA = 0.1
_BETA = math.log(0.5 / 1 + 1.0)

_N_NODES = 10000
_HIDDEN = 128

_NC = 2
_NS = 16
_NW = _NC * _NS
_CHUNK = 16
_CHUNKS_PER_W = 640
_EDGES_PER_W = _CHUNK * _CHUNKS_PER_W
_EP = _NW * _EDGES_PER_W
_NBUF = 5
_ROWS_PER_TILE = 640
_ZCOPY = 16


def _sc_aggregate(xb, ei, nrm):
    mesh = plsc.VectorSubcoreMesh(core_axis_name="c", subcore_axis_name="s")

    def body(x_hbm, ei_hbm, nrm_hbm, out_hbm, rows5, stag, iring, nring,
             acc, g0, g1, g2, g3, g4, e0, e1, e2, e3, e4, s0, s1):
        c = lax.axis_index("c")
        s = lax.axis_index("s")
        wid = s * _NC + c
        gsems = (g0, g1, g2, g3, g4)
        esems = (e0, e1, e2, e3, e4, e0, e1, e2, e3, e4)
        ssems = (s0, s1)

        def ecopy(k, slot):
            pltpu.async_copy(ei_hbm.at[0, wid, k], iring.at[slot, 0],
                             esems[slot])
            pltpu.async_copy(ei_hbm.at[1, wid, k], iring.at[slot, 1],
                             esems[slot])
            pltpu.async_copy(nrm_hbm.at[wid, k], nring.at[slot],
                             esems[slot])

        def ewait(k, slot):
            pltpu.make_async_copy(ei_hbm.at[0, wid, k], iring.at[slot, 0],
                                  esems[slot]).wait()
            pltpu.make_async_copy(ei_hbm.at[1, wid, k], iring.at[slot, 1],
                                  esems[slot]).wait()
            pltpu.make_async_copy(nrm_hbm.at[wid, k], nring.at[slot],
                                  esems[slot]).wait()

        def gather(islot, rslot):
            pltpu.async_copy(x_hbm.at[iring.at[islot, 0]], rows5.at[rslot],
                             gsems[rslot])

        def gwait(islot, rslot):
            pltpu.make_async_copy(x_hbm.at[iring.at[islot, 0]],
                                  rows5.at[rslot], gsems[rslot]).wait()

        def scatter(islot, h):
            pltpu.async_copy(stag.at[h], acc.at[iring.at[islot, 1]],
                             ssems[h], add=True)

        def swait(islot, h):
            pltpu.make_async_copy(stag.at[h], acc.at[iring.at[islot, 1]],
                                  ssems[h]).wait()

        def scale(islot, rslot, h):
            n16 = nring[islot, pl.ds(0, 16)]
            for jj in range(16):
                nb = jnp.full((16,), n16[jj], jnp.float32)
                for f in range(_HIDDEN // 32):
                    w = rows5[rslot, jj, pl.ds(f * 16, 16)]
                    lo = lax.bitcast_convert_type(w << 16, jnp.float32)
                    hi = lax.bitcast_convert_type(
                        w & jnp.full((16,), -65536, jnp.int32),
                        jnp.float32)
                    stag[h, jj, pl.ds(f * 32, 16)] = lo * nb
                    stag[h, jj, pl.ds(f * 32 + 16, 16)] = hi * nb

        def zrow(e, carry):
            for f in range(_HIDDEN // 16):
                stag[0, e, pl.ds(f * 16, 16)] = jnp.zeros((16,), jnp.float32)
            return carry
        lax.fori_loop(0, _CHUNK, zrow, 0)
        base = s * _ROWS_PER_TILE

        @pl.when(s < _NS - 1)
        def _():
            for i in range(_ROWS_PER_TILE // _ZCOPY):
                pltpu.sync_copy(stag.at[0],
                                acc.at[pl.ds(base + i * _ZCOPY, _ZCOPY)])

        @pl.when(s == _NS - 1)
        def _():
            for i in range(25):
                pltpu.sync_copy(
                    stag.at[0], acc.at[pl.ds(9600 + i * _ZCOPY, _ZCOPY)])
        plsc.subcore_barrier()

        for kk in range(4):
            ecopy(kk, kk)
        for kk in range(3):
            ewait(kk, kk)
            gather(kk, kk)

        _M = _CHUNKS_PER_W // 10
        def ring(m, carry):
            for j in range(10):
                rslot = j % _NBUF
                h = j % 2
                gwait(j, rslot)
                gi = (j + 3) % 10
                gr = (j + 3) % _NBUF
                if j < 7:
                    ewait(10 * m + j + 3, gi)
                    gather(gi, gr)
                else:
                    @pl.when(m < _M - 1)
                    def _():
                        ewait(10 * m + j + 3, gi)
                        gather(gi, gr)
                if j < 6:
                    ecopy(10 * m + j + 4, (j + 4) % 10)
                else:
                    @pl.when(m < _M - 1)
                    def _():
                        ecopy(10 * m + j + 4, (j + 4) % 10)
                scale(j, rslot, h)
                if j == 0:
                    @pl.when(m >= 1)
                    def _():
                        swait(9, 1)
                else:
                    swait(j - 1, (j - 1) % 2)
                scatter(j, h)
            return carry
        lax.fori_loop(0, _M, ring, 0)

        swait(9, 1)

        plsc.subcore_barrier()

        @pl.when(s < _NS - 1)
        def _():
            pltpu.sync_copy(acc.at[pl.ds(base, _ROWS_PER_TILE)],
                            out_hbm.at[c, pl.ds(base, _ROWS_PER_TILE)])

        @pl.when(s == _NS - 1)
        def _():
            pltpu.sync_copy(acc.at[pl.ds(9600, 400)],
                            out_hbm.at[c, pl.ds(9600, 400)])

    return pl.kernel(
        body,
        out_type=jax.ShapeDtypeStruct((_NC, _N_NODES, _HIDDEN), jnp.float32),
        mesh=mesh,
        compiler_params=pltpu.CompilerParams(use_tc_tiling_on_sc=False),
        scratch_types=[
            pltpu.VMEM((_NBUF, _CHUNK, _HIDDEN // 2), jnp.int32),
            pltpu.VMEM((2, _CHUNK, _HIDDEN), jnp.float32),
            pltpu.VMEM((10, 2, _CHUNK), jnp.int32),
            pltpu.VMEM((10, _CHUNK), jnp.float32),
            pltpu.VMEM_SHARED((_N_NODES, _HIDDEN), jnp.float32),
        ] + [pltpu.SemaphoreType.DMA] * 12,
    )(xb, ei, nrm)


def _combine_body(p_ref, x0_ref, w_ref, o_ref):
    p = p_ref[...]
    h = (1.0 - _ALPHA) * (p[0] + p[1]) + _ALPHA * x0_ref[...]
    hw = lax.dot_general(h, w_ref[...], (((1,), (1,)), ((), ())),
                         preferred_element_type=jnp.float32)
    o_ref[...] = (1.0 - _BETA) * h + _BETA * hw


def _tc_combine(partials, x0, W):
    blk = 400
    grid = _N_NODES // blk
    return pl.pallas_call(
        _combine_body,
        grid=(grid,),
        in_specs=[
            pl.BlockSpec((_NC, blk, _HIDDEN), lambda i: (0, i, 0)),
            pl.BlockSpec((blk, _HIDDEN), lambda i: (i, 0)),
            pl.BlockSpec((_HIDDEN, _HIDDEN), lambda i: (0, 0)),
        ],
        out_specs=pl.BlockSpec((blk, _HIDDEN), lambda i: (i, 0)),
        out_shape=jax.ShapeDtypeStruct((_N_NODES, _HIDDEN), jnp.float32),
    )(partials, x0, W)


def kernel(x, x0, edge_index, norm, W):
    e = edge_index.shape[1]
    pad = _EP - e
    ei = jnp.concatenate(
        [edge_index.astype(jnp.int32),
         jnp.zeros((2, pad), jnp.int32)], axis=1).reshape(
        2, _NW, _CHUNKS_PER_W, _CHUNK)
    nrm = jnp.concatenate(
        [norm.astype(jnp.float32), jnp.zeros((pad,), jnp.float32)]).reshape(
        _NW, _CHUNKS_PER_W, _CHUNK)
    xb = lax.bitcast_convert_type(
        x.astype(jnp.bfloat16).reshape(
            _N_NODES, _HIDDEN // 32, 2, 16).transpose(0, 1, 3, 2),
        jnp.int32).reshape(_N_NODES, _HIDDEN // 2)
    partials = _sc_aggregate(xb, ei, nrm)
    return _tc_combine(partials, x0, W)


# speedup vs baseline: 1.0259x; 1.0259x over previous
import math

import jax
import jax.numpy as jnp
from jax import lax
from jax.experimental import pallas as pl
from jax.experimental.pallas import tpu as pltpu
from jax.experimental.pallas import tpu_sc as plsc

_ALPHA = 0.1
_BETA = math.log(0.5 / 1 + 1.0)

_N_NODES = 10000
_HIDDEN = 128

_NC = 2
_NS = 16
_NW = _NC * _NS
_CHUNK = 16
_CHUNKS_PER_W = 625
_EDGES_PER_W = _CHUNK * _CHUNKS_PER_W
_NBUF = 5
_ROWS_PER_TILE = 640
_ZCOPY = 16


def _sc_aggregate(xb, ei, nrm):
    mesh = plsc.VectorSubcoreMesh(core_axis_name="c", subcore_axis_name="s")

    def body(x_hbm, ei_hbm, nrm_hbm, out_hbm, rows5, stag, iring, nring,
             acc, xs, g0, g1, g2, g3, g4, e0, e1, e2, e3, e4):
        c = lax.axis_index("c")
        s = lax.axis_index("s")
        wid = s * _NC + c
        gsems = (g0, g1, g2, g3, g4)
        esems = (e0, e1, e2, e3, e4)

        def ecopy(k, slot):
            pltpu.async_copy(ei_hbm.at[0, wid, k], iring.at[slot, 0],
                             esems[slot])
            pltpu.async_copy(ei_hbm.at[1, wid, k], iring.at[slot, 1],
                             esems[slot])
            pltpu.async_copy(nrm_hbm.at[wid, k], nring.at[slot],
                             esems[slot])

        def ewait(k, slot):
            pltpu.make_async_copy(ei_hbm.at[0, wid, k], iring.at[slot, 0],
                                  esems[slot]).wait()
            pltpu.make_async_copy(ei_hbm.at[1, wid, k], iring.at[slot, 1],
                                  esems[slot]).wait()
            pltpu.make_async_copy(nrm_hbm.at[wid, k], nring.at[slot],
                                  esems[slot]).wait()

        def gather(slot):
            pltpu.async_copy(xs.at[iring.at[slot, 0]], rows5.at[slot],
                             gsems[slot])

        def gwait(slot):
            pltpu.make_async_copy(xs.at[iring.at[slot, 0]],
                                  rows5.at[slot], gsems[slot]).wait()

        def scatter(slot):
            pltpu.sync_copy(stag, acc.at[iring.at[slot, 1]], add=True)

        def scale(slot):
            def sg(g, c2):
                n16 = nring[slot, pl.ds(g * 16, 16)]
                for jj in range(16):
                    e = g * 16 + jj
                    nb = jnp.full((16,), n16[jj], jnp.float32)
                    for f in range(_HIDDEN // 32):
                        w = rows5[slot, e, pl.ds(f * 16, 16)]
                        lo = lax.bitcast_convert_type(w << 16, jnp.float32)
                        hi = lax.bitcast_convert_type(
                            w & jnp.full((16,), -65536, jnp.int32),
                            jnp.float32)
                        stag[e, pl.ds(f * 32, 16)] = lo * nb
                        stag[e, pl.ds(f * 32 + 16, 16)] = hi * nb
                return c2
            lax.fori_loop(0, _CHUNK // 16, sg, 0)

        def zrow(e, carry):
            for f in range(_HIDDEN // 16):
                stag[e, pl.ds(f * 16, 16)] = jnp.zeros((16,), jnp.float32)
            return carry
        lax.fori_loop(0, _CHUNK, zrow, 0)
        base = s * _ROWS_PER_TILE

        @pl.when(s < _NS - 1)
        def _():
            for i in range(_ROWS_PER_TILE // _ZCOPY):
                pltpu.sync_copy(stag, acc.at[pl.ds(base + i * _ZCOPY,
                                                   _ZCOPY)])

        @pl.when(s == _NS - 1)
        def _():
            for i in range(25):
                pltpu.sync_copy(
                    stag, acc.at[pl.ds(9600 + i * _ZCOPY, _ZCOPY)])

        pltpu.sync_copy(x_hbm.at[pl.ds(s * 625, 625)],
                        xs.at[pl.ds(s * 625, 625)])
        plsc.subcore_barrier()

        for kk in range(4):
            ecopy(kk, kk)
        for kk in range(3):
            ewait(kk, kk)
            gather(kk)

        _M = _CHUNKS_PER_W // _NBUF
        def ring(m, carry):
            for j in range(_NBUF):
                gwait(j)
                gslot = (j + 3) % _NBUF
                if j < 2:
                    ewait(5 * m + j + 3, gslot)
                    gather(gslot)
                else:
                    @pl.when(m < _M - 1)
                    def _():
                        ewait(5 * m + j + 3, gslot)
                        gather(gslot)
                if j < 1:
                    ecopy(5 * m + j + 4, (j + 4) % _NBUF)
                else:
                    @pl.when(m < _M - 1)
                    def _():
                        ecopy(5 * m + j + 4, (j + 4) % _NBUF)
                scale(j)
                scatter(j)
            return carry
        lax.fori_loop(0, _M, ring, 0)

        plsc.subcore_barrier()

        @pl.when(s < _NS - 1)
        def _():
            pltpu.sync_copy(acc.at[pl.ds(base, _ROWS_PER_TILE)],
                            out_hbm.at[c, pl.ds(base, _ROWS_PER_TILE)])

        @pl.when(s == _NS - 1)
        def _():
            pltpu.sync_copy(acc.at[pl.ds(9600, 400)],
                            out_hbm.at[c, pl.ds(9600, 400)])

    return pl.kernel(
        body,
        out_type=jax.ShapeDtypeStruct((_NC, _N_NODES, _HIDDEN), jnp.float32),
        mesh=mesh,
        compiler_params=pltpu.CompilerParams(use_tc_tiling_on_sc=False),
        scratch_types=[
            pltpu.VMEM((_NBUF, _CHUNK, _HIDDEN // 2), jnp.int32),
            pltpu.VMEM((_CHUNK, _HIDDEN), jnp.float32),
            pltpu.VMEM((_NBUF, 2, _CHUNK), jnp.int32),
            pltpu.VMEM((_NBUF, _CHUNK), jnp.float32),
            pltpu.VMEM_SHARED((_N_NODES, _HIDDEN), jnp.float32),
            pltpu.VMEM_SHARED((_N_NODES, _HIDDEN // 2), jnp.int32),
        ] + [pltpu.SemaphoreType.DMA] * 10,
    )(xb, ei, nrm)


def _combine_body(p_ref, x0_ref, w_ref, o_ref):
    p = p_ref[...]
    h = (1.0 - _ALPHA) * (p[0] + p[1]) + _ALPHA * x0_ref[...]
    hw = lax.dot_general(h, w_ref[...], (((1,), (1,)), ((), ())),
                         preferred_element_type=jnp.float32)
    o_ref[...] = (1.0 - _BETA) * h + _BETA * hw


def _tc_combine(partials, x0, W):
    blk = 400
    grid = _N_NODES // blk
    return pl.pallas_call(
        _combine_body,
        grid=(grid,),
        in_specs=[
            pl.BlockSpec((_NC, blk, _HIDDEN), lambda i: (0, i, 0)),
            pl.BlockSpec((blk, _HIDDEN), lambda i: (i, 0)),
            pl.BlockSpec((_HIDDEN, _HIDDEN), lambda i: (0, 0)),
        ],
        out_specs=pl.BlockSpec((blk, _HIDDEN), lambda i: (i, 0)),
        out_shape=jax.ShapeDtypeStruct((_N_NODES, _HIDDEN), jnp.float32),
    )(partials, x0, W)


def kernel(x, x0, edge_index, norm, W):
    ei = edge_index.astype(jnp.int32).reshape(
        2, _NW, _CHUNKS_PER_W, _CHUNK)
    nrm = norm.astype(jnp.float32).reshape(_NW, _CHUNKS_PER_W, _CHUNK)
    xb = lax.bitcast_convert_type(
        x.astype(jnp.bfloat16).reshape(
            _N_NODES, _HIDDEN // 32, 2, 16).transpose(0, 1, 3, 2),
        jnp.int32).reshape(_N_NODES, _HIDDEN // 2)
    partials = _sc_aggregate(xb, ei, nrm)
    return _tc_combine(partials, x0, W)
